# Initial kernel scaffold; baseline (speedup 1.0000x reference)
import jax
import jax.numpy as jnp
from jax.experimental import pallas as pl


def kernel(x, edge_index, pW1, pb1, pW2, pb2, cW1, cb1, cW2, cb2, lW, lb):
    raise NotImplementedError("write your pallas kernel here")


# speedup vs baseline: 6.8445x; 6.8445x over previous
import functools

import jax
import jax.numpy as jnp
from jax import lax
from jax.experimental import pallas as pl
from jax.experimental.pallas import tpu as pltpu
from jax.experimental.pallas import tpu_sc as plsc

NC = 2
NS = 16
IB = 80
SUB = 5
CHUNK = IB * SUB
IB1 = 40
SUB1 = 5
CHUNK1 = IB1 * SUB1


def _mesh():
    return plsc.VectorSubcoreMesh(
        core_axis_name="c", subcore_axis_name="s", num_cores=NC, num_subcores=NS
    )


def _k1_call(a, b, src3, dst3, w2, pb2v, zn, n, e):
    ew = e // (NC * NS)
    nchunks = ew // CHUNK1

    def body(a_hbm, b_hbm, src_hbm, dst_hbm, w2_hbm, pb2_hbm, zn_hbm,
             w_hbm, deg_hbm,
             a_sh, b_sh, deg_sh,
             src_v, dst_v, a_rows, b_rows, w_buf, p_buf, w2_v, pb2_v, sem):
        c = lax.axis_index("c")
        s = lax.axis_index("s")
        wid = s * NC + c

        @pl.when(s == 0)
        def _stage():
            pltpu.sync_copy(a_hbm, a_sh)
            pltpu.sync_copy(b_hbm, b_sh)
            pltpu.sync_copy(zn_hbm, deg_sh)

        pltpu.sync_copy(w2_hbm, w2_v)
        pltpu.sync_copy(pb2_hbm, pb2_v)
        plsc.subcore_barrier()

        iot = lax.iota(jnp.int32, 16)

        def chunk(k, carry):
            pltpu.sync_copy(src_hbm.at[wid, pl.ds(k * SUB1, SUB1)], src_v)
            pltpu.sync_copy(dst_hbm.at[wid, pl.ds(k * SUB1, SUB1)], dst_v)
            for j in range(SUB1):
                cp1 = pltpu.async_copy(
                    a_sh.at[src_v.at[j]],
                    a_rows.at[pl.ds(j * IB1, IB1)], sem)
                cp2 = pltpu.async_copy(
                    b_sh.at[dst_v.at[j]],
                    b_rows.at[pl.ds(j * IB1, IB1)], sem)
                cp1.wait()
                cp2.wait()

            def grp(g, carry2):
                for j in range(16):
                    eidx = g * 16 + j
                    acc = None
                    for q in range(4):
                        zv = (a_rows[eidx, pl.ds(q * 16, 16)]
                              + b_rows[eidx, pl.ds(q * 16, 16)])
                        hv = jnp.maximum(zv, 0.0) * w2_v[pl.ds(q * 16, 16)]
                        acc = hv if acc is None else acc + hv
                    p_buf[pl.ds(j * 16, 16)] = acc
                tot = pb2_v[...]
                for l in range(16):
                    tot = tot + plsc.load_gather(p_buf, [iot * 16 + l])
                w_buf[pl.ds(g * 16, 16)] = 1.0 / (1.0 + jnp.exp(-tot))
                return carry2

            lax.fori_loop(0, CHUNK1 // 16, grp, 0)
            ebase = wid * ew + k * CHUNK1
            pltpu.sync_copy(w_buf, w_hbm.at[pl.ds(ebase, CHUNK1)])
            for j in range(SUB1):
                pltpu.sync_copy(w_buf.at[pl.ds(j * IB1, IB1)],
                                deg_sh.at[dst_v.at[j]], add=True)
            return carry

        lax.fori_loop(0, nchunks, chunk, 0)
        plsc.subcore_barrier()

        @pl.when(s == 0)
        def _out():
            pltpu.sync_copy(deg_sh, deg_hbm.at[pl.ds(c * n, n)])

    f = pl.kernel(
        body,
        out_type=(jax.ShapeDtypeStruct((e,), jnp.float32),
                  jax.ShapeDtypeStruct((NC * n,), jnp.float32)),
        mesh=_mesh(),
        compiler_params=pltpu.CompilerParams(needs_layout_passes=False, use_tc_tiling_on_sc=False),
        scratch_types=[
            pltpu.VMEM_SHARED((n, 64), jnp.float32),
            pltpu.VMEM_SHARED((n, 64), jnp.float32),
            pltpu.VMEM_SHARED((n,), jnp.float32),
            pltpu.VMEM((SUB1, IB1), jnp.int32),
            pltpu.VMEM((SUB1, IB1), jnp.int32),
            pltpu.VMEM((CHUNK1, 64), jnp.float32),
            pltpu.VMEM((CHUNK1, 64), jnp.float32),
            pltpu.VMEM((CHUNK1,), jnp.float32),
            pltpu.VMEM((256,), jnp.float32),
            pltpu.VMEM((64,), jnp.float32),
            pltpu.VMEM((16,), jnp.float32),
            pltpu.SemaphoreType.DMA,
        ],
    )
    return f(a, b, src3, dst3, w2, pb2v, zn)


def _k2a_call(xws, dis, w, src2, dst2, z64, n, e):
    ew = e // NS
    nchunks = ew // CHUNK

    def body(xws_hbm, dis_hbm, w_hbm, src_hbm, dst_hbm, z64_hbm,
             agg_hbm, norm_hbm,
             xw_sh, dis_sh, acc_sh,
             src_v, dst_v, x_rows, wc_v, ds_v, dd_v, nrm_v, sem):
        c = lax.axis_index("c")
        s = lax.axis_index("s")

        @pl.when(s == 0)
        def _stage():
            pltpu.sync_copy(xws_hbm.at[c], xw_sh)
            pltpu.sync_copy(dis_hbm, dis_sh)
            pltpu.sync_copy(z64_hbm, acc_sh)

        plsc.subcore_barrier()

        def chunk(k, carry):
            ebase = s * ew + k * CHUNK
            pltpu.sync_copy(src_hbm.at[s, pl.ds(k * SUB, SUB)], src_v)
            pltpu.sync_copy(dst_hbm.at[s, pl.ds(k * SUB, SUB)], dst_v)
            pltpu.sync_copy(w_hbm.at[pl.ds(ebase, CHUNK)], wc_v)
            for j in range(SUB):
                cp1 = pltpu.async_copy(
                    dis_sh.at[src_v.at[j]],
                    ds_v.at[pl.ds(j * IB, IB)], sem)
                cp2 = pltpu.async_copy(
                    dis_sh.at[dst_v.at[j]],
                    dd_v.at[pl.ds(j * IB, IB)], sem)
                cp3 = pltpu.async_copy(
                    xw_sh.at[src_v.at[j]],
                    x_rows.at[pl.ds(j * IB, IB)], sem)
                cp1.wait()
                cp2.wait()
                cp3.wait()
            for t in range(CHUNK // 16):
                sl = pl.ds(t * 16, 16)
                nrm_v[sl] = ds_v[sl] * wc_v[sl] * dd_v[sl]

            @pl.when(c == 0)
            def _wnorm():
                pltpu.sync_copy(nrm_v, norm_hbm.at[pl.ds(ebase, CHUNK)])

            def grp(g, carry2):
                for j in range(16):
                    eidx = g * 16 + j
                    nb = plsc.load_gather(
                        nrm_v, [jnp.full((16,), eidx, jnp.int32)])
                    for q in range(4):
                        sl = pl.ds(q * 16, 16)
                        x_rows[eidx, sl] = x_rows[eidx, sl] * nb
                return carry2

            lax.fori_loop(0, CHUNK // 16, grp, 0)
            for j in range(SUB):
                pltpu.sync_copy(x_rows.at[pl.ds(j * IB, IB)],
                                acc_sh.at[dst_v.at[j]], add=True)
            return carry

        lax.fori_loop(0, nchunks, chunk, 0)
        plsc.subcore_barrier()

        @pl.when(s == 0)
        def _out():
            pltpu.sync_copy(acc_sh, agg_hbm.at[c])

    f = pl.kernel(
        body,
        out_type=(jax.ShapeDtypeStruct((NC, n, 64), jnp.float32),
                  jax.ShapeDtypeStruct((e,), jnp.float32)),
        mesh=_mesh(),
        compiler_params=pltpu.CompilerParams(needs_layout_passes=False, use_tc_tiling_on_sc=False),
        scratch_types=[
            pltpu.VMEM_SHARED((n, 64), jnp.float32),
            pltpu.VMEM_SHARED((n,), jnp.float32),
            pltpu.VMEM_SHARED((n, 64), jnp.float32),
            pltpu.VMEM((SUB, IB), jnp.int32),
            pltpu.VMEM((SUB, IB), jnp.int32),
            pltpu.VMEM((CHUNK, 64), jnp.float32),
            pltpu.VMEM((CHUNK,), jnp.float32),
            pltpu.VMEM((CHUNK,), jnp.float32),
            pltpu.VMEM((CHUNK,), jnp.float32),
            pltpu.VMEM((CHUNK,), jnp.float32),
            pltpu.SemaphoreType.DMA,
        ],
    )
    return f(xws, dis, w, src2, dst2, z64)


def _k2b_call(xws, norm, src2, dst2, z64, n, e):
    ew = e // NS
    nchunks = ew // CHUNK

    def body(xws_hbm, norm_hbm, src_hbm, dst_hbm, z64_hbm,
             agg_hbm,
             xw_sh, acc_sh,
             src_v, dst_v, x_rows, nrm_v, sem):
        c = lax.axis_index("c")
        s = lax.axis_index("s")

        @pl.when(s == 0)
        def _stage():
            pltpu.sync_copy(xws_hbm.at[c], xw_sh)
            pltpu.sync_copy(z64_hbm, acc_sh)

        plsc.subcore_barrier()

        def chunk(k, carry):
            ebase = s * ew + k * CHUNK
            pltpu.sync_copy(src_hbm.at[s, pl.ds(k * SUB, SUB)], src_v)
            pltpu.sync_copy(dst_hbm.at[s, pl.ds(k * SUB, SUB)], dst_v)
            pltpu.sync_copy(norm_hbm.at[pl.ds(ebase, CHUNK)], nrm_v)
            for j in range(SUB):
                pltpu.async_copy(
                    xw_sh.at[src_v.at[j]],
                    x_rows.at[pl.ds(j * IB, IB)], sem).wait()

            def grp(g, carry2):
                for j in range(16):
                    eidx = g * 16 + j
                    nb = plsc.load_gather(
                        nrm_v, [jnp.full((16,), eidx, jnp.int32)])
                    for q in range(4):
                        sl = pl.ds(q * 16, 16)
                        x_rows[eidx, sl] = x_rows[eidx, sl] * nb
                return carry2

            lax.fori_loop(0, CHUNK // 16, grp, 0)
            for j in range(SUB):
                pltpu.sync_copy(x_rows.at[pl.ds(j * IB, IB)],
                                acc_sh.at[dst_v.at[j]], add=True)
            return carry

        lax.fori_loop(0, nchunks, chunk, 0)
        plsc.subcore_barrier()

        @pl.when(s == 0)
        def _out():
            pltpu.sync_copy(acc_sh, agg_hbm.at[c])

    f = pl.kernel(
        body,
        out_type=jax.ShapeDtypeStruct((NC, n, 64), jnp.float32),
        mesh=_mesh(),
        compiler_params=pltpu.CompilerParams(needs_layout_passes=False, use_tc_tiling_on_sc=False),
        scratch_types=[
            pltpu.VMEM_SHARED((n, 64), jnp.float32),
            pltpu.VMEM_SHARED((n, 64), jnp.float32),
            pltpu.VMEM((SUB, IB), jnp.int32),
            pltpu.VMEM((SUB, IB), jnp.int32),
            pltpu.VMEM((CHUNK, 64), jnp.float32),
            pltpu.VMEM((CHUNK,), jnp.float32),
            pltpu.SemaphoreType.DMA,
        ],
    )
    return f(xws, norm, src2, dst2, z64)


_BR = 400


def _p0_call(x, wa, wb, w1t, pb1r, n, d):
    def body(x_ref, wa_ref, wb_ref, w1_ref, pb1_ref, a_ref, b_ref, xw_ref):
        xb = x_ref[...]
        a_ref[...] = jnp.dot(xb, wa_ref[...],
                             preferred_element_type=jnp.float32) + pb1_ref[...]
        b_ref[...] = jnp.dot(xb, wb_ref[...],
                             preferred_element_type=jnp.float32)
        xw_ref[...] = jnp.dot(xb, w1_ref[...],
                              preferred_element_type=jnp.float32)

    grid = (n // _BR,)
    return pl.pallas_call(
        body,
        grid=grid,
        in_specs=[
            pl.BlockSpec((_BR, d), lambda i: (i, 0)),
            pl.BlockSpec((d, 64), lambda i: (0, 0)),
            pl.BlockSpec((d, 64), lambda i: (0, 0)),
            pl.BlockSpec((d, d), lambda i: (0, 0)),
            pl.BlockSpec((1, 64), lambda i: (0, 0)),
        ],
        out_specs=[
            pl.BlockSpec((_BR, 64), lambda i: (i, 0)),
            pl.BlockSpec((_BR, 64), lambda i: (i, 0)),
            pl.BlockSpec((_BR, d), lambda i: (i, 0)),
        ],
        out_shape=[
            jax.ShapeDtypeStruct((n, 64), jnp.float32),
            jax.ShapeDtypeStruct((n, 64), jnp.float32),
            jax.ShapeDtypeStruct((n, d), jnp.float32),
        ],
    )(x, wa, wb, w1t, pb1r)


def _p2_call(deg2, n):
    def body(deg_ref, dis_ref, invd_ref):
        deg = deg_ref[0:1, :] + deg_ref[1:2, :] + 1.0
        dis_ref[...] = lax.rsqrt(deg)
        invd_ref[...] = 1.0 / deg

    return pl.pallas_call(
        body,
        out_shape=[
            jax.ShapeDtypeStruct((1, n), jnp.float32),
            jax.ShapeDtypeStruct((1, n), jnp.float32),
        ],
    )(deg2)


def _p4_call(a0, a1, xw, invd, cbr, w2t, n, d):
    def body(a0_ref, a1_ref, xw_ref, iv_ref, cb_ref, w_ref, o_ref):
        x1 = jnp.concatenate([a0_ref[...], a1_ref[...]], axis=1)
        x1 = x1 + xw_ref[...] * iv_ref[...] + cb_ref[...]
        x1 = jnp.maximum(x1, 0.0)
        o_ref[...] = jnp.dot(x1, w_ref[...], preferred_element_type=jnp.float32)

    grid = (n // _BR,)
    return pl.pallas_call(
        body,
        grid=grid,
        in_specs=[
            pl.BlockSpec((_BR, 64), lambda i: (i, 0)),
            pl.BlockSpec((_BR, 64), lambda i: (i, 0)),
            pl.BlockSpec((_BR, d), lambda i: (i, 0)),
            pl.BlockSpec((_BR, 1), lambda i: (i, 0)),
            pl.BlockSpec((1, d), lambda i: (0, 0)),
            pl.BlockSpec((d, d), lambda i: (0, 0)),
        ],
        out_specs=pl.BlockSpec((_BR, d), lambda i: (i, 0)),
        out_shape=jax.ShapeDtypeStruct((n, d), jnp.float32),
    )(a0, a1, xw, invd, cbr, w2t)


def _p6_call(a0, a1, xw, invd, cbr, lwt, lbr, n, d, o):
    def body(a0_ref, a1_ref, xw_ref, iv_ref, cb_ref, w_ref, b_ref, o_ref):
        x2 = jnp.concatenate([a0_ref[...], a1_ref[...]], axis=1)
        x2 = x2 + xw_ref[...] * iv_ref[...] + cb_ref[...]
        x2 = jnp.maximum(x2, 0.0)
        z = jnp.dot(x2, w_ref[...], preferred_element_type=jnp.float32)
        o_ref[...] = 1.0 / (1.0 + jnp.exp(-(z + b_ref[...])))

    grid = (n // _BR,)
    return pl.pallas_call(
        body,
        grid=grid,
        in_specs=[
            pl.BlockSpec((_BR, 64), lambda i: (i, 0)),
            pl.BlockSpec((_BR, 64), lambda i: (i, 0)),
            pl.BlockSpec((_BR, d), lambda i: (i, 0)),
            pl.BlockSpec((_BR, 1), lambda i: (i, 0)),
            pl.BlockSpec((1, d), lambda i: (0, 0)),
            pl.BlockSpec((d, o), lambda i: (0, 0)),
            pl.BlockSpec((1, o), lambda i: (0, 0)),
        ],
        out_specs=pl.BlockSpec((_BR, o), lambda i: (i, 0)),
        out_shape=jax.ShapeDtypeStruct((n, o), jnp.float32),
    )(a0, a1, xw, invd, cbr, lwt, lbr)


def kernel(x, edge_index, pW1, pb1, pW2, pb2, cW1, cb1, cW2, cb2, lW, lb):
    n, d = x.shape
    e = edge_index.shape[1]
    o = lW.shape[0]

    src = edge_index[0].astype(jnp.int32)
    dst = edge_index[1].astype(jnp.int32)
    src1 = src.reshape(NC * NS, e // (NC * NS * IB1), IB1)
    dst1 = dst.reshape(NC * NS, e // (NC * NS * IB1), IB1)
    src2 = src.reshape(NS, e // (NS * IB), IB)
    dst2 = dst.reshape(NS, e // (NS * IB), IB)

    wa = pW1[:, :d].T
    wb = pW1[:, d:].T
    w1t = cW1.T
    w2t = cW2.T
    lwt = lW.T
    pb1r = pb1.reshape(1, 64)
    pb2v = jnp.broadcast_to(pb2.reshape(1), (16,))
    w2v = pW2.reshape(64)
    zn = jnp.zeros((n,), jnp.float32)
    z64 = jnp.zeros((n, 64), jnp.float32)

    a, b, xw1 = _p0_call(x, wa, wb, w1t, pb1r, n, d)
    w, deg_flat = _k1_call(a, b, src1, dst1, w2v, pb2v, zn, n, e)
    dis, invd = _p2_call(deg_flat.reshape(NC, n), n)
    dis = dis.reshape(n)
    invd_c = invd.reshape(n, 1)

    xw1s = xw1.reshape(n, NC, 64).transpose(1, 0, 2)
    agg1, norm = _k2a_call(xw1s, dis, w, src2, dst2, z64, n, e)
    xw2 = _p4_call(agg1[0], agg1[1], xw1, invd_c, cb1.reshape(1, d), w2t, n, d)

    xw2s = xw2.reshape(n, NC, 64).transpose(1, 0, 2)
    agg2 = _k2b_call(xw2s, norm, src2, dst2, z64, n, e)
    out = _p6_call(agg2[0], agg2[1], xw2, invd_c, cb2.reshape(1, d), lwt,
                   lb.reshape(1, o), n, d, o)
    return out


# speedup vs baseline: 7.9570x; 1.1625x over previous
import functools

import jax
import jax.numpy as jnp
from jax import lax
from jax.experimental import pallas as pl
from jax.experimental.pallas import tpu as pltpu
from jax.experimental.pallas import tpu_sc as plsc

NC = 2
NS = 16
IB = 80
SUB = 5
CHUNK = IB * SUB
IB1 = 80
SUB1 = 5
CHUNK1 = IB1 * SUB1


def _mesh():
    return plsc.VectorSubcoreMesh(
        core_axis_name="c", subcore_axis_name="s", num_cores=NC, num_subcores=NS
    )


def _k1_call(a, b, src3, dst3, w2, pb2v, zn, n, e):
    ew = e // (NC * NS)
    nchunks = ew // CHUNK1
    npairs = nchunks // 2
    odd = nchunks % 2 == 1

    def body(a_hbm, b_hbm, src_hbm, dst_hbm, w2_hbm, pb2_hbm, zn_hbm,
             w_hbm, deg_hbm,
             deg_sh,
             src_a, dst_a, src_b, dst_b,
             arows_a, brows_a, arows_b, brows_b,
             w_buf, p_buf, w2_v, pb2_v, sem_a, sem_b):
        c = lax.axis_index("c")
        s = lax.axis_index("s")
        wid = s * NC + c

        @pl.when(s == 0)
        def _stage():
            pltpu.sync_copy(zn_hbm, deg_sh)

        pltpu.sync_copy(w2_hbm, w2_v)
        pltpu.sync_copy(pb2_hbm, pb2_v)
        plsc.subcore_barrier()

        iot = lax.iota(jnp.int32, 16)

        def load_idx(cidx, sv, dv):
            pltpu.sync_copy(src_hbm.at[wid, pl.ds(cidx * SUB1, SUB1)], sv)
            pltpu.sync_copy(dst_hbm.at[wid, pl.ds(cidx * SUB1, SUB1)], dv)

        def fire(sv, dv, arows, brows, sem):
            for j in range(SUB1):
                pltpu.async_copy(
                    a_hbm.at[sv.at[j]], arows.at[pl.ds(j * IB1, IB1)], sem)
                pltpu.async_copy(
                    b_hbm.at[dv.at[j]], brows.at[pl.ds(j * IB1, IB1)], sem)

        def drain(sv, dv, arows, brows, sem):
            for j in range(SUB1):
                pltpu.make_async_copy(
                    a_hbm.at[sv.at[j]], arows.at[pl.ds(j * IB1, IB1)],
                    sem).wait()
                pltpu.make_async_copy(
                    b_hbm.at[dv.at[j]], brows.at[pl.ds(j * IB1, IB1)],
                    sem).wait()

        def compute(cidx, arows, brows, dv):
            def grp(g, carry2):
                for j in range(16):
                    eidx = g * 16 + j
                    acc = None
                    for q in range(4):
                        zv = (arows[eidx, pl.ds(q * 16, 16)]
                              + brows[eidx, pl.ds(q * 16, 16)])
                        hv = jnp.maximum(zv, 0.0) * w2_v[pl.ds(q * 16, 16)]
                        acc = hv if acc is None else acc + hv
                    p_buf[pl.ds(j * 16, 16)] = acc
                tot = pb2_v[...]
                for l in range(16):
                    tot = tot + plsc.load_gather(p_buf, [iot * 16 + l])
                w_buf[pl.ds(g * 16, 16)] = 1.0 / (1.0 + jnp.exp(-tot))
                return carry2

            lax.fori_loop(0, CHUNK1 // 16, grp, 0)
            ebase = wid * ew + cidx * CHUNK1
            pltpu.sync_copy(w_buf, w_hbm.at[pl.ds(ebase, CHUNK1)])
            for j in range(SUB1):
                pltpu.sync_copy(w_buf.at[pl.ds(j * IB1, IB1)],
                                deg_sh.at[dv.at[j]], add=True)

        load_idx(0, src_a, dst_a)
        fire(src_a, dst_a, arows_a, brows_a, sem_a)

        def pair(k, carry):
            cb = 2 * k + 1
            load_idx(cb, src_b, dst_b)
            fire(src_b, dst_b, arows_b, brows_b, sem_b)
            drain(src_a, dst_a, arows_a, brows_a, sem_a)
            compute(2 * k, arows_a, brows_a, dst_a)
            ca2 = 2 * k + 2

            @pl.when(ca2 < nchunks)
            def _next():
                load_idx(ca2, src_a, dst_a)
                fire(src_a, dst_a, arows_a, brows_a, sem_a)

            drain(src_b, dst_b, arows_b, brows_b, sem_b)
            compute(cb, arows_b, brows_b, dst_b)
            return carry

        lax.fori_loop(0, npairs, pair, 0)
        if odd:
            drain(src_a, dst_a, arows_a, brows_a, sem_a)
            compute(nchunks - 1, arows_a, brows_a, dst_a)
        plsc.subcore_barrier()

        @pl.when(s == 0)
        def _out():
            pltpu.sync_copy(deg_sh, deg_hbm.at[pl.ds(c * n, n)])

    f = pl.kernel(
        body,
        out_type=(jax.ShapeDtypeStruct((e,), jnp.float32),
                  jax.ShapeDtypeStruct((NC * n,), jnp.float32)),
        mesh=_mesh(),
        compiler_params=pltpu.CompilerParams(
            needs_layout_passes=False, use_tc_tiling_on_sc=False),
        scratch_types=[
            pltpu.VMEM_SHARED((n,), jnp.float32),
            pltpu.VMEM((SUB1, IB1), jnp.int32),
            pltpu.VMEM((SUB1, IB1), jnp.int32),
            pltpu.VMEM((SUB1, IB1), jnp.int32),
            pltpu.VMEM((SUB1, IB1), jnp.int32),
            pltpu.VMEM((CHUNK1, 64), jnp.float32),
            pltpu.VMEM((CHUNK1, 64), jnp.float32),
            pltpu.VMEM((CHUNK1, 64), jnp.float32),
            pltpu.VMEM((CHUNK1, 64), jnp.float32),
            pltpu.VMEM((CHUNK1,), jnp.float32),
            pltpu.VMEM((256,), jnp.float32),
            pltpu.VMEM((64,), jnp.float32),
            pltpu.VMEM((16,), jnp.float32),
            pltpu.SemaphoreType.DMA,
            pltpu.SemaphoreType.DMA,
        ],
    )
    return f(a, b, src3, dst3, w2, pb2v, zn)


def _k2_call(xws, w, src2, dst2, z64, n, e):
    ew = e // NS
    nchunks = ew // CHUNK
    npairs = nchunks // 2
    odd = nchunks % 2 == 1

    def body(xws_hbm, w_hbm, src_hbm, dst_hbm, z64_hbm,
             agg_hbm,
             acc_sh,
             src_a, dst_a, src_b, dst_b,
             rows_a, rows_b, wc_v, sem_a, sem_b):
        c = lax.axis_index("c")
        s = lax.axis_index("s")
        tab = xws_hbm.at[c]

        @pl.when(s == 0)
        def _stage():
            pltpu.sync_copy(z64_hbm, acc_sh)

        plsc.subcore_barrier()

        def load_idx(cidx, sv, dv):
            pltpu.sync_copy(src_hbm.at[s, pl.ds(cidx * SUB, SUB)], sv)
            pltpu.sync_copy(dst_hbm.at[s, pl.ds(cidx * SUB, SUB)], dv)

        def fire(sv, rows, sem):
            for j in range(SUB):
                pltpu.async_copy(
                    tab.at[sv.at[j]], rows.at[pl.ds(j * IB, IB)], sem)

        def drain(sv, rows, sem):
            for j in range(SUB):
                pltpu.make_async_copy(
                    tab.at[sv.at[j]], rows.at[pl.ds(j * IB, IB)], sem).wait()

        def compute(cidx, rows, dv):
            ebase = s * ew + cidx * CHUNK
            pltpu.sync_copy(w_hbm.at[pl.ds(ebase, CHUNK)], wc_v)

            def grp(g, carry2):
                for j in range(16):
                    eidx = g * 16 + j
                    nb = plsc.load_gather(
                        wc_v, [jnp.full((16,), eidx, jnp.int32)])
                    for q in range(4):
                        sl = pl.ds(q * 16, 16)
                        rows[eidx, sl] = rows[eidx, sl] * nb
                return carry2

            lax.fori_loop(0, CHUNK // 16, grp, 0)
            for j in range(SUB):
                pltpu.sync_copy(rows.at[pl.ds(j * IB, IB)],
                                acc_sh.at[dv.at[j]], add=True)

        load_idx(0, src_a, dst_a)
        fire(src_a, rows_a, sem_a)

        def pair(k, carry):
            cb = 2 * k + 1
            load_idx(cb, src_b, dst_b)
            fire(src_b, rows_b, sem_b)
            drain(src_a, rows_a, sem_a)
            compute(2 * k, rows_a, dst_a)
            ca2 = 2 * k + 2

            @pl.when(ca2 < nchunks)
            def _next():
                load_idx(ca2, src_a, dst_a)
                fire(src_a, rows_a, sem_a)

            drain(src_b, rows_b, sem_b)
            compute(cb, rows_b, dst_b)
            return carry

        lax.fori_loop(0, npairs, pair, 0)
        if odd:
            drain(src_a, rows_a, sem_a)
            compute(nchunks - 1, rows_a, dst_a)
        plsc.subcore_barrier()

        @pl.when(s == 0)
        def _out():
            pltpu.sync_copy(acc_sh, agg_hbm.at[c])

    f = pl.kernel(
        body,
        out_type=jax.ShapeDtypeStruct((NC, n, 64), jnp.float32),
        mesh=_mesh(),
        compiler_params=pltpu.CompilerParams(
            needs_layout_passes=False, use_tc_tiling_on_sc=False),
        scratch_types=[
            pltpu.VMEM_SHARED((n, 64), jnp.float32),
            pltpu.VMEM((SUB, IB), jnp.int32),
            pltpu.VMEM((SUB, IB), jnp.int32),
            pltpu.VMEM((SUB, IB), jnp.int32),
            pltpu.VMEM((SUB, IB), jnp.int32),
            pltpu.VMEM((CHUNK, 64), jnp.float32),
            pltpu.VMEM((CHUNK, 64), jnp.float32),
            pltpu.VMEM((CHUNK,), jnp.float32),
            pltpu.SemaphoreType.DMA,
            pltpu.SemaphoreType.DMA,
        ],
    )
    return f(xws, w, src2, dst2, z64)


_BR = 400


def _p0_call(x, wa, wb, w1t, pb1r, n, d):
    def body(x_ref, wa_ref, wb_ref, w1_ref, pb1_ref, a_ref, b_ref, xw_ref):
        xb = x_ref[...]
        a_ref[...] = jnp.dot(xb, wa_ref[...],
                             preferred_element_type=jnp.float32) + pb1_ref[...]
        b_ref[...] = jnp.dot(xb, wb_ref[...],
                             preferred_element_type=jnp.float32)
        xw_ref[...] = jnp.dot(xb, w1_ref[...],
                              preferred_element_type=jnp.float32)

    grid = (n // _BR,)
    return pl.pallas_call(
        body,
        grid=grid,
        in_specs=[
            pl.BlockSpec((_BR, d), lambda i: (i, 0)),
            pl.BlockSpec((d, 64), lambda i: (0, 0)),
            pl.BlockSpec((d, 64), lambda i: (0, 0)),
            pl.BlockSpec((d, d), lambda i: (0, 0)),
            pl.BlockSpec((1, 64), lambda i: (0, 0)),
        ],
        out_specs=[
            pl.BlockSpec((_BR, 64), lambda i: (i, 0)),
            pl.BlockSpec((_BR, 64), lambda i: (i, 0)),
            pl.BlockSpec((_BR, d), lambda i: (i, 0)),
        ],
        out_shape=[
            jax.ShapeDtypeStruct((n, 64), jnp.float32),
            jax.ShapeDtypeStruct((n, 64), jnp.float32),
            jax.ShapeDtypeStruct((n, d), jnp.float32),
        ],
    )(x, wa, wb, w1t, pb1r)


def _p2_call(deg2, xw1, n, d):
    def body(deg_ref, xw_ref, dis_ref, xwd_ref):
        deg = deg_ref[:, 0:1] + deg_ref[:, 1:2] + 1.0
        dis = lax.rsqrt(deg)
        dis_ref[...] = dis
        xwd_ref[...] = xw_ref[...] * dis

    grid = (n // _BR,)
    return pl.pallas_call(
        body,
        grid=grid,
        in_specs=[
            pl.BlockSpec((_BR, 2), lambda i: (i, 0)),
            pl.BlockSpec((_BR, d), lambda i: (i, 0)),
        ],
        out_specs=[
            pl.BlockSpec((_BR, 1), lambda i: (i, 0)),
            pl.BlockSpec((_BR, d), lambda i: (i, 0)),
        ],
        out_shape=[
            jax.ShapeDtypeStruct((n, 1), jnp.float32),
            jax.ShapeDtypeStruct((n, d), jnp.float32),
        ],
    )(deg2, xw1)


def _p4_call(a0, a1, xwd, dis, cbr, w2t, n, d):
    def body(a0_ref, a1_ref, xwd_ref, dis_ref, cb_ref, w_ref, o_ref):
        agg = jnp.concatenate([a0_ref[...], a1_ref[...]], axis=1)
        x1 = (agg + xwd_ref[...]) * dis_ref[...] + cb_ref[...]
        x1 = jnp.maximum(x1, 0.0)
        o_ref[...] = jnp.dot(
            x1, w_ref[...], preferred_element_type=jnp.float32) * dis_ref[...]

    grid = (n // _BR,)
    return pl.pallas_call(
        body,
        grid=grid,
        in_specs=[
            pl.BlockSpec((_BR, 64), lambda i: (i, 0)),
            pl.BlockSpec((_BR, 64), lambda i: (i, 0)),
            pl.BlockSpec((_BR, d), lambda i: (i, 0)),
            pl.BlockSpec((_BR, 1), lambda i: (i, 0)),
            pl.BlockSpec((1, d), lambda i: (0, 0)),
            pl.BlockSpec((d, d), lambda i: (0, 0)),
        ],
        out_specs=pl.BlockSpec((_BR, d), lambda i: (i, 0)),
        out_shape=jax.ShapeDtypeStruct((n, d), jnp.float32),
    )(a0, a1, xwd, dis, cbr, w2t)


def _p6_call(a0, a1, xwd, dis, cbr, lwt, lbr, n, d, o):
    def body(a0_ref, a1_ref, xwd_ref, dis_ref, cb_ref, w_ref, b_ref, o_ref):
        agg = jnp.concatenate([a0_ref[...], a1_ref[...]], axis=1)
        x2 = (agg + xwd_ref[...]) * dis_ref[...] + cb_ref[...]
        x2 = jnp.maximum(x2, 0.0)
        z = jnp.dot(x2, w_ref[...], preferred_element_type=jnp.float32)
        o_ref[...] = 1.0 / (1.0 + jnp.exp(-(z + b_ref[...])))

    grid = (n // _BR,)
    return pl.pallas_call(
        body,
        grid=grid,
        in_specs=[
            pl.BlockSpec((_BR, 64), lambda i: (i, 0)),
            pl.BlockSpec((_BR, 64), lambda i: (i, 0)),
            pl.BlockSpec((_BR, d), lambda i: (i, 0)),
            pl.BlockSpec((_BR, 1), lambda i: (i, 0)),
            pl.BlockSpec((1, d), lambda i: (0, 0)),
            pl.BlockSpec((d, o), lambda i: (0, 0)),
            pl.BlockSpec((1, o), lambda i: (0, 0)),
        ],
        out_specs=pl.BlockSpec((_BR, o), lambda i: (i, 0)),
        out_shape=jax.ShapeDtypeStruct((n, o), jnp.float32),
    )(a0, a1, xwd, dis, cbr, lwt, lbr)


def kernel(x, edge_index, pW1, pb1, pW2, pb2, cW1, cb1, cW2, cb2, lW, lb):
    n, d = x.shape
    e = edge_index.shape[1]
    o = lW.shape[0]

    src = edge_index[0].astype(jnp.int32)
    dst = edge_index[1].astype(jnp.int32)
    src1 = src.reshape(NC * NS, e // (NC * NS * IB1), IB1)
    dst1 = dst.reshape(NC * NS, e // (NC * NS * IB1), IB1)
    src2 = src.reshape(NS, e // (NS * IB), IB)
    dst2 = dst.reshape(NS, e // (NS * IB), IB)

    wa = pW1[:, :d].T
    wb = pW1[:, d:].T
    w1t = cW1.T
    w2t = cW2.T
    lwt = lW.T
    pb1r = pb1.reshape(1, 64)
    pb2v = jnp.broadcast_to(pb2.reshape(1), (16,))
    w2v = pW2.reshape(64)
    zn = jnp.zeros((n,), jnp.float32)
    z64 = jnp.zeros((n, 64), jnp.float32)

    a, b, xw1 = _p0_call(x, wa, wb, w1t, pb1r, n, d)
    w, deg_flat = _k1_call(a, b, src1, dst1, w2v, pb2v, zn, n, e)
    dis_c, xw1d = _p2_call(deg_flat.reshape(NC, n).T, xw1, n, d)

    xw1s = xw1d.reshape(n, NC, 64).transpose(1, 0, 2)
    agg1 = _k2_call(xw1s, w, src2, dst2, z64, n, e)
    xw2d = _p4_call(agg1[0], agg1[1], xw1d, dis_c, cb1.reshape(1, d), w2t,
                    n, d)

    xw2s = xw2d.reshape(n, NC, 64).transpose(1, 0, 2)
    agg2 = _k2_call(xw2s, w, src2, dst2, z64, n, e)
    out = _p6_call(agg2[0], agg2[1], xw2d, dis_c, cb2.reshape(1, d), lwt,
                   lb.reshape(1, o), n, d, o)
    return out


# speedup vs baseline: 9.2600x; 1.1637x over previous
import functools

import jax
import jax.numpy as jnp
from jax import lax
from jax.experimental import pallas as pl
from jax.experimental.pallas import tpu as pltpu
from jax.experimental.pallas import tpu_sc as plsc

NC = 2
NS = 16
IB = 400
SUB = 1
CHUNK = IB * SUB
IB1 = 400
SUB1 = 1
CHUNK1 = IB1 * SUB1


def _mesh():
    return plsc.VectorSubcoreMesh(
        core_axis_name="c", subcore_axis_name="s", num_cores=NC, num_subcores=NS
    )


def _k1_call(a, b, src3, dst3, w2, pb2v, zn, n, e):
    ew = e // (NC * NS)
    nchunks = ew // CHUNK1
    npairs = nchunks // 2
    odd = nchunks % 2 == 1

    def body(a_hbm, b_hbm, src_hbm, dst_hbm, w2_hbm, pb2_hbm, zn_hbm,
             w_hbm, deg_hbm,
             deg_sh,
             src_a, dst_a, src_b, dst_b,
             arows_a, brows_a, arows_b, brows_b,
             w_buf, p_buf, w2_v, pb2_v, sem_a, sem_b):
        c = lax.axis_index("c")
        s = lax.axis_index("s")
        wid = s * NC + c

        @pl.when(s == 0)
        def _stage():
            pltpu.sync_copy(zn_hbm, deg_sh)

        pltpu.sync_copy(w2_hbm, w2_v)
        pltpu.sync_copy(pb2_hbm, pb2_v)
        plsc.subcore_barrier()

        iot = lax.iota(jnp.int32, 16)

        def load_idx(cidx, sv, dv):
            pltpu.sync_copy(src_hbm.at[wid, pl.ds(cidx * SUB1, SUB1)], sv)
            pltpu.sync_copy(dst_hbm.at[wid, pl.ds(cidx * SUB1, SUB1)], dv)

        def fire(sv, dv, arows, brows, sem):
            for j in range(SUB1):
                pltpu.async_copy(
                    a_hbm.at[sv.at[j]], arows.at[pl.ds(j * IB1, IB1)], sem)
                pltpu.async_copy(
                    b_hbm.at[dv.at[j]], brows.at[pl.ds(j * IB1, IB1)], sem)

        def drain(sv, dv, arows, brows, sem):
            for j in range(SUB1):
                pltpu.make_async_copy(
                    a_hbm.at[sv.at[j]], arows.at[pl.ds(j * IB1, IB1)],
                    sem).wait()
                pltpu.make_async_copy(
                    b_hbm.at[dv.at[j]], brows.at[pl.ds(j * IB1, IB1)],
                    sem).wait()

        def compute(cidx, arows, brows, dv):
            def grp(g, carry2):
                for j in range(16):
                    eidx = g * 16 + j
                    acc = None
                    for q in range(4):
                        zv = (arows[eidx, pl.ds(q * 16, 16)]
                              + brows[eidx, pl.ds(q * 16, 16)])
                        hv = jnp.maximum(zv, 0.0) * w2_v[pl.ds(q * 16, 16)]
                        acc = hv if acc is None else acc + hv
                    p_buf[pl.ds(j * 16, 16)] = acc
                tot = pb2_v[...]
                for l in range(16):
                    tot = tot + plsc.load_gather(p_buf, [iot * 16 + l])
                w_buf[pl.ds(g * 16, 16)] = 1.0 / (1.0 + jnp.exp(-tot))
                return carry2

            lax.fori_loop(0, CHUNK1 // 16, grp, 0)
            ebase = wid * ew + cidx * CHUNK1
            pltpu.sync_copy(w_buf, w_hbm.at[pl.ds(ebase, CHUNK1)])
            for j in range(SUB1):
                pltpu.sync_copy(w_buf.at[pl.ds(j * IB1, IB1)],
                                deg_sh.at[dv.at[j]], add=True)

        load_idx(0, src_a, dst_a)
        fire(src_a, dst_a, arows_a, brows_a, sem_a)

        def pair(k, carry):
            cb = 2 * k + 1
            load_idx(cb, src_b, dst_b)
            fire(src_b, dst_b, arows_b, brows_b, sem_b)
            drain(src_a, dst_a, arows_a, brows_a, sem_a)
            compute(2 * k, arows_a, brows_a, dst_a)
            ca2 = 2 * k + 2

            @pl.when(ca2 < nchunks)
            def _next():
                load_idx(ca2, src_a, dst_a)
                fire(src_a, dst_a, arows_a, brows_a, sem_a)

            drain(src_b, dst_b, arows_b, brows_b, sem_b)
            compute(cb, arows_b, brows_b, dst_b)
            return carry

        lax.fori_loop(0, npairs, pair, 0)
        if odd:
            drain(src_a, dst_a, arows_a, brows_a, sem_a)
            compute(nchunks - 1, arows_a, brows_a, dst_a)
        plsc.subcore_barrier()

        @pl.when(s == 0)
        def _out():
            pltpu.sync_copy(deg_sh, deg_hbm.at[pl.ds(c * n, n)])

    f = pl.kernel(
        body,
        out_type=(jax.ShapeDtypeStruct((e,), jnp.float32),
                  jax.ShapeDtypeStruct((NC * n,), jnp.float32)),
        mesh=_mesh(),
        compiler_params=pltpu.CompilerParams(
            needs_layout_passes=False, use_tc_tiling_on_sc=False),
        scratch_types=[
            pltpu.VMEM_SHARED((n,), jnp.float32),
            pltpu.VMEM((SUB1, IB1), jnp.int32),
            pltpu.VMEM((SUB1, IB1), jnp.int32),
            pltpu.VMEM((SUB1, IB1), jnp.int32),
            pltpu.VMEM((SUB1, IB1), jnp.int32),
            pltpu.VMEM((CHUNK1, 64), jnp.float32),
            pltpu.VMEM((CHUNK1, 64), jnp.float32),
            pltpu.VMEM((CHUNK1, 64), jnp.float32),
            pltpu.VMEM((CHUNK1, 64), jnp.float32),
            pltpu.VMEM((CHUNK1,), jnp.float32),
            pltpu.VMEM((256,), jnp.float32),
            pltpu.VMEM((64,), jnp.float32),
            pltpu.VMEM((16,), jnp.float32),
            pltpu.SemaphoreType.DMA,
            pltpu.SemaphoreType.DMA,
        ],
    )
    return f(a, b, src3, dst3, w2, pb2v, zn)


def _k2_call(xws, w, src2, dst2, z64, n, e):
    ew = e // NS
    nchunks = ew // CHUNK
    npairs = nchunks // 2
    assert nchunks % 2 == 0

    def body(xws_hbm, w_hbm, src_hbm, dst_hbm, z64_hbm,
             agg_hbm,
             acc_sh,
             src_a, dst_a, src_b, dst_b,
             rows_a, rows_b, wc_a, wc_b,
             sem_ga, sem_gb, sem_sa, sem_sb,
             sem_ia, sem_ib, sem_da, sem_db):
        c = lax.axis_index("c")
        s = lax.axis_index("s")
        tab = xws_hbm.at[c]

        @pl.when(s == 0)
        def _stage():
            pltpu.sync_copy(z64_hbm, acc_sh)

        plsc.subcore_barrier()

        def fire_src(cidx, sv, wv, sem):
            pltpu.async_copy(src_hbm.at[s, pl.ds(cidx * SUB, SUB)], sv, sem)
            pltpu.async_copy(
                w_hbm.at[pl.ds(s * ew + cidx * CHUNK, CHUNK)], wv, sem)

        def drain_src(cidx, sv, wv, sem):
            pltpu.make_async_copy(
                src_hbm.at[s, pl.ds(cidx * SUB, SUB)], sv, sem).wait()
            pltpu.make_async_copy(
                w_hbm.at[pl.ds(s * ew + cidx * CHUNK, CHUNK)], wv, sem).wait()

        def fire_dst(cidx, dv, sem):
            pltpu.async_copy(dst_hbm.at[s, pl.ds(cidx * SUB, SUB)], dv, sem)

        def drain_dst(cidx, dv, sem):
            pltpu.make_async_copy(
                dst_hbm.at[s, pl.ds(cidx * SUB, SUB)], dv, sem).wait()

        def fire_gather(sv, rows, sem):
            for j in range(SUB):
                pltpu.async_copy(
                    tab.at[sv.at[j]], rows.at[pl.ds(j * IB, IB)], sem)

        def drain_gather(sv, rows, sem):
            for j in range(SUB):
                pltpu.make_async_copy(
                    tab.at[sv.at[j]], rows.at[pl.ds(j * IB, IB)], sem).wait()

        def fire_scatter(dv, rows, sem):
            for j in range(SUB):
                pltpu.async_copy(
                    rows.at[pl.ds(j * IB, IB)], acc_sh.at[dv.at[j]], sem)

        def drain_scatter(dv, rows, sem):
            for j in range(SUB):
                pltpu.make_async_copy(
                    rows.at[pl.ds(j * IB, IB)], acc_sh.at[dv.at[j]],
                    sem).wait()

        def compute(rows, wv):
            def grp(g, carry2):
                for j in range(16):
                    eidx = g * 16 + j
                    nb = plsc.load_gather(
                        wv, [jnp.full((16,), eidx, jnp.int32)])
                    for q in range(4):
                        sl = pl.ds(q * 16, 16)
                        rows[eidx, sl] = rows[eidx, sl] * nb
                return carry2

            lax.fori_loop(0, CHUNK // 16, grp, 0)

        pltpu.sync_copy(src_hbm.at[s, pl.ds(0, SUB)], src_a)
        pltpu.sync_copy(w_hbm.at[pl.ds(s * ew, CHUNK)], wc_a)
        fire_gather(src_a, rows_a, sem_ga)
        fire_dst(0, dst_a, sem_da)
        fire_src(1, src_b, wc_b, sem_ib)

        def pair(k, carry):
            ca = 2 * k
            cb = 2 * k + 1
            drain_src(cb, src_b, wc_b, sem_ib)

            @pl.when(k > 0)
            def _dsb():
                drain_scatter(dst_b, rows_b, sem_sb)

            fire_gather(src_b, rows_b, sem_gb)
            fire_dst(cb, dst_b, sem_db)
            drain_gather(src_a, rows_a, sem_ga)
            compute(rows_a, wc_a)

            @pl.when(k < npairs - 1)
            def _fia():
                fire_src(ca + 2, src_a, wc_a, sem_ia)

            drain_dst(ca, dst_a, sem_da)
            fire_scatter(dst_a, rows_a, sem_sa)
            drain_gather(src_b, rows_b, sem_gb)
            compute(rows_b, wc_b)

            @pl.when(k < npairs - 1)
            def _fib():
                fire_src(cb + 2, src_b, wc_b, sem_ib)

            drain_dst(cb, dst_b, sem_db)
            fire_scatter(dst_b, rows_b, sem_sb)

            @pl.when(k < npairs - 1)
            def _nga():
                drain_scatter(dst_a, rows_a, sem_sa)
                drain_src(ca + 2, src_a, wc_a, sem_ia)
                fire_gather(src_a, rows_a, sem_ga)
                fire_dst(ca + 2, dst_a, sem_da)

            return carry

        lax.fori_loop(0, npairs, pair, 0)
        drain_scatter(dst_a, rows_a, sem_sa)
        drain_scatter(dst_b, rows_b, sem_sb)
        plsc.subcore_barrier()

        @pl.when(s == 0)
        def _out():
            pltpu.sync_copy(acc_sh, agg_hbm.at[c])

    f = pl.kernel(
        body,
        out_type=jax.ShapeDtypeStruct((NC, n, 64), jnp.float32),
        mesh=_mesh(),
        compiler_params=pltpu.CompilerParams(
            needs_layout_passes=False, use_tc_tiling_on_sc=False),
        scratch_types=[
            pltpu.VMEM_SHARED((n, 64), jnp.float32),
            pltpu.VMEM((SUB, IB), jnp.int32),
            pltpu.VMEM((SUB, IB), jnp.int32),
            pltpu.VMEM((SUB, IB), jnp.int32),
            pltpu.VMEM((SUB, IB), jnp.int32),
            pltpu.VMEM((CHUNK, 64), jnp.float32),
            pltpu.VMEM((CHUNK, 64), jnp.float32),
            pltpu.VMEM((CHUNK,), jnp.float32),
            pltpu.VMEM((CHUNK,), jnp.float32),
            pltpu.SemaphoreType.DMA,
            pltpu.SemaphoreType.DMA,
            pltpu.SemaphoreType.DMA,
            pltpu.SemaphoreType.DMA,
            pltpu.SemaphoreType.DMA,
            pltpu.SemaphoreType.DMA,
            pltpu.SemaphoreType.DMA,
            pltpu.SemaphoreType.DMA,
        ],
    )
    return f(xws, w, src2, dst2, z64)


_BR = 400


def _p0_call(x, wa, wb, w1t, pb1r, n, d):
    def body(x_ref, wa_ref, wb_ref, w1_ref, pb1_ref, a_ref, b_ref, xw_ref):
        xb = x_ref[...]
        a_ref[...] = jnp.dot(xb, wa_ref[...],
                             preferred_element_type=jnp.float32) + pb1_ref[...]
        b_ref[...] = jnp.dot(xb, wb_ref[...],
                             preferred_element_type=jnp.float32)
        xw_ref[...] = jnp.dot(xb, w1_ref[...],
                              preferred_element_type=jnp.float32)

    grid = (n // _BR,)
    return pl.pallas_call(
        body,
        grid=grid,
        in_specs=[
            pl.BlockSpec((_BR, d), lambda i: (i, 0)),
            pl.BlockSpec((d, 64), lambda i: (0, 0)),
            pl.BlockSpec((d, 64), lambda i: (0, 0)),
            pl.BlockSpec((d, d), lambda i: (0, 0)),
            pl.BlockSpec((1, 64), lambda i: (0, 0)),
        ],
        out_specs=[
            pl.BlockSpec((_BR, 64), lambda i: (i, 0)),
            pl.BlockSpec((_BR, 64), lambda i: (i, 0)),
            pl.BlockSpec((_BR, d), lambda i: (i, 0)),
        ],
        out_shape=[
            jax.ShapeDtypeStruct((n, 64), jnp.float32),
            jax.ShapeDtypeStruct((n, 64), jnp.float32),
            jax.ShapeDtypeStruct((n, d), jnp.float32),
        ],
    )(x, wa, wb, w1t, pb1r)


def _p2_call(deg2, xw1, n, d):
    def body(deg_ref, xw_ref, dis_ref, xwd_ref):
        deg = deg_ref[:, 0:1] + deg_ref[:, 1:2] + 1.0
        dis = lax.rsqrt(deg)
        dis_ref[...] = dis
        xwd_ref[...] = xw_ref[...] * dis

    grid = (n // _BR,)
    return pl.pallas_call(
        body,
        grid=grid,
        in_specs=[
            pl.BlockSpec((_BR, 2), lambda i: (i, 0)),
            pl.BlockSpec((_BR, d), lambda i: (i, 0)),
        ],
        out_specs=[
            pl.BlockSpec((_BR, 1), lambda i: (i, 0)),
            pl.BlockSpec((_BR, d), lambda i: (i, 0)),
        ],
        out_shape=[
            jax.ShapeDtypeStruct((n, 1), jnp.float32),
            jax.ShapeDtypeStruct((n, d), jnp.float32),
        ],
    )(deg2, xw1)


def _p4_call(a0, a1, xwd, dis, cbr, w2t, n, d):
    def body(a0_ref, a1_ref, xwd_ref, dis_ref, cb_ref, w_ref, o_ref):
        agg = jnp.concatenate([a0_ref[...], a1_ref[...]], axis=1)
        x1 = (agg + xwd_ref[...]) * dis_ref[...] + cb_ref[...]
        x1 = jnp.maximum(x1, 0.0)
        o_ref[...] = jnp.dot(
            x1, w_ref[...], preferred_element_type=jnp.float32) * dis_ref[...]

    grid = (n // _BR,)
    return pl.pallas_call(
        body,
        grid=grid,
        in_specs=[
            pl.BlockSpec((_BR, 64), lambda i: (i, 0)),
            pl.BlockSpec((_BR, 64), lambda i: (i, 0)),
            pl.BlockSpec((_BR, d), lambda i: (i, 0)),
            pl.BlockSpec((_BR, 1), lambda i: (i, 0)),
            pl.BlockSpec((1, d), lambda i: (0, 0)),
            pl.BlockSpec((d, d), lambda i: (0, 0)),
        ],
        out_specs=pl.BlockSpec((_BR, d), lambda i: (i, 0)),
        out_shape=jax.ShapeDtypeStruct((n, d), jnp.float32),
    )(a0, a1, xwd, dis, cbr, w2t)


def _p6_call(a0, a1, xwd, dis, cbr, lwt, lbr, n, d, o):
    def body(a0_ref, a1_ref, xwd_ref, dis_ref, cb_ref, w_ref, b_ref, o_ref):
        agg = jnp.concatenate([a0_ref[...], a1_ref[...]], axis=1)
        x2 = (agg + xwd_ref[...]) * dis_ref[...] + cb_ref[...]
        x2 = jnp.maximum(x2, 0.0)
        z = jnp.dot(x2, w_ref[...], preferred_element_type=jnp.float32)
        o_ref[...] = 1.0 / (1.0 + jnp.exp(-(z + b_ref[...])))

    grid = (n // _BR,)
    return pl.pallas_call(
        body,
        grid=grid,
        in_specs=[
            pl.BlockSpec((_BR, 64), lambda i: (i, 0)),
            pl.BlockSpec((_BR, 64), lambda i: (i, 0)),
            pl.BlockSpec((_BR, d), lambda i: (i, 0)),
            pl.BlockSpec((_BR, 1), lambda i: (i, 0)),
            pl.BlockSpec((1, d), lambda i: (0, 0)),
            pl.BlockSpec((d, o), lambda i: (0, 0)),
            pl.BlockSpec((1, o), lambda i: (0, 0)),
        ],
        out_specs=pl.BlockSpec((_BR, o), lambda i: (i, 0)),
        out_shape=jax.ShapeDtypeStruct((n, o), jnp.float32),
    )(a0, a1, xwd, dis, cbr, lwt, lbr)


def kernel(x, edge_index, pW1, pb1, pW2, pb2, cW1, cb1, cW2, cb2, lW, lb):
    n, d = x.shape
    e = edge_index.shape[1]
    o = lW.shape[0]

    src = edge_index[0].astype(jnp.int32)
    dst = edge_index[1].astype(jnp.int32)
    src1 = src.reshape(NC * NS, e // (NC * NS * IB1), IB1)
    dst1 = dst.reshape(NC * NS, e // (NC * NS * IB1), IB1)
    src2 = src.reshape(NS, e // (NS * IB), IB)
    dst2 = dst.reshape(NS, e // (NS * IB), IB)

    wa = pW1[:, :d].T
    wb = pW1[:, d:].T
    w1t = cW1.T
    w2t = cW2.T
    lwt = lW.T
    pb1r = pb1.reshape(1, 64)
    pb2v = jnp.broadcast_to(pb2.reshape(1), (16,))
    w2v = pW2.reshape(64)
    zn = jnp.zeros((n,), jnp.float32)
    z64 = jnp.zeros((n, 64), jnp.float32)

    a, b, xw1 = _p0_call(x, wa, wb, w1t, pb1r, n, d)
    w, deg_flat = _k1_call(a, b, src1, dst1, w2v, pb2v, zn, n, e)
    dis_c, xw1d = _p2_call(deg_flat.reshape(NC, n).T, xw1, n, d)

    xw1s = xw1d.reshape(n, NC, 64).transpose(1, 0, 2)
    agg1 = _k2_call(xw1s, w, src2, dst2, z64, n, e)
    xw2d = _p4_call(agg1[0], agg1[1], xw1d, dis_c, cb1.reshape(1, d), w2t,
                    n, d)

    xw2s = xw2d.reshape(n, NC, 64).transpose(1, 0, 2)
    agg2 = _k2_call(xw2s, w, src2, dst2, z64, n, e)
    out = _p6_call(agg2[0], agg2[1], xw2d, dis_c, cb2.reshape(1, d), lwt,
                   lb.reshape(1, o), n, d, o)
    return out
